# Initial kernel scaffold; baseline (speedup 1.0000x reference)
import jax
import jax.numpy as jnp
from jax.experimental import pallas as pl


def kernel(x, edge_attr, edge_index, batch, params):
    raise NotImplementedError("write your pallas kernel here")


# speedup vs baseline: 1.4660x; 1.4660x over previous
import functools
import jax
import jax.numpy as jnp
from jax import lax
from jax.experimental import pallas as pl
from jax.experimental.pallas import tpu as pltpu
from jax.experimental.pallas import tpu_sc as plsc

NN = 10000
NE = 160000
HID = 128
NG = 64
NP_ = 10240
EP_ = 163840

NC, NS = 2, 16
NW = NC * NS
E_PER_W = EP_ // NW
CH = 128
NCH = E_PER_W // CH
ROWS_PER_SUB = NP_ // NS


@functools.lru_cache(maxsize=None)
def _sc_kernels():
    mesh = plsc.VectorSubcoreMesh(core_axis_name="c", subcore_axis_name="s")

    @functools.partial(
        pl.kernel, mesh=mesh,
        out_type=[jax.ShapeDtypeStruct((EP_, HID), jnp.float32),
                  jax.ShapeDtypeStruct((EP_, HID), jnp.float32)],
        scratch_types=[
            pltpu.VMEM((CH,), jnp.int32),
            pltpu.VMEM((CH,), jnp.int32),
            pltpu.VMEM((CH, HID), jnp.float32),
            pltpu.VMEM((CH, HID), jnp.float32),
            pltpu.SemaphoreType.DMA,
            pltpu.SemaphoreType.DMA,
        ],
    )
    def gather_k(x_hbm, dst_hbm, src_hbm, od_hbm, os_hbm,
                 idxd_v, idxs_v, rowsd_v, rowss_v, semd, sems):
        wid = lax.axis_index("s") * NC + lax.axis_index("c")
        base = wid * E_PER_W

        def body(i, c):
            off = base + i * CH
            pltpu.sync_copy(dst_hbm.at[pl.ds(off, CH)], idxd_v)
            pltpu.sync_copy(src_hbm.at[pl.ds(off, CH)], idxs_v)
            cpd = pltpu.async_copy(x_hbm.at[idxd_v], rowsd_v, semd)
            cps = pltpu.async_copy(x_hbm.at[idxs_v], rowss_v, sems)
            cpd.wait()
            cps.wait()
            pltpu.sync_copy(rowsd_v, od_hbm.at[pl.ds(off, CH)])
            pltpu.sync_copy(rowss_v, os_hbm.at[pl.ds(off, CH)])
            return c

        lax.fori_loop(0, NCH, body, 0)

    @functools.partial(
        pl.kernel, mesh=mesh,
        out_type=jax.ShapeDtypeStruct((2 * NP_, HID), jnp.float32),
        scratch_types=[
            pltpu.VMEM((CH,), jnp.int32),
            pltpu.VMEM((CH, HID), jnp.float32),
            pltpu.VMEM_SHARED((NP_, HID), jnp.float32),
            pltpu.SemaphoreType.DMA,
        ],
    )
    def scatter_k(msg_hbm, dst_hbm, zeros_hbm, out_hbm, idx_v, msg_v, acc_sh, sem):
        cid = lax.axis_index("c")
        sid = lax.axis_index("s")
        wid = sid * NC + cid
        base = wid * E_PER_W
        pltpu.sync_copy(zeros_hbm.at[pl.ds(sid * ROWS_PER_SUB, ROWS_PER_SUB)],
                        acc_sh.at[pl.ds(sid * ROWS_PER_SUB, ROWS_PER_SUB)])
        plsc.subcore_barrier()

        def body(i, c):
            off = base + i * CH
            pltpu.sync_copy(dst_hbm.at[pl.ds(off, CH)], idx_v)
            pltpu.sync_copy(msg_hbm.at[pl.ds(off, CH)], msg_v)
            pltpu.sync_copy(msg_v, acc_sh.at[idx_v], add=True)
            return c

        lax.fori_loop(0, NCH, body, 0)
        plsc.subcore_barrier()
        pltpu.sync_copy(acc_sh.at[pl.ds(sid * ROWS_PER_SUB, ROWS_PER_SUB)],
                        out_hbm.at[pl.ds(cid * NP_ + sid * ROWS_PER_SUB, ROWS_PER_SUB)])

    return gather_k, scatter_k


def _sc_gather(x, dst, src):
    return _sc_kernels()[0](x, dst, src)


def _sc_scatter(msg, dst, zeros_nodes):
    return _sc_kernels()[1](msg, dst, zeros_nodes)


BN = 512
BE = 1024


def _dot(a, b):
    return lax.dot_general(a, b, (((a.ndim - 1,), (0,)), ((), ())),
                           preferred_element_type=jnp.float32)


def _init_body(x_ref, wemb_ref, bemb_ref, xo_ref):
    xo_ref[...] = jnp.maximum(_dot(x_ref[...], wemb_ref[...]) + bemb_ref[...], 0.0)


def _edge_body(ea_ref, xd_ref, xs_ref, wemb_ref, bemb_ref, w_ref, b_ref, msg_ref):
    ea = jnp.maximum(_dot(ea_ref[...], wemb_ref[...]) + bemb_ref[...], 0.0)
    z = jnp.concatenate([xd_ref[...], xs_ref[...], ea], axis=1)
    g = _dot(z, w_ref[...]) + b_ref[...]
    gate = jax.nn.sigmoid(g[:, :HID])
    core = jax.nn.softplus(g[:, HID:])
    msg_ref[...] = gate * core


def _node_body(x_ref, a0_ref, a1_ref, gam_ref, bet_ref, mu_ref, var_ref,
               xo_ref, *, residual):
    x = x_ref[...]
    h = x + (a0_ref[...] + a1_ref[...])
    h = (h - mu_ref[...]) / jnp.sqrt(var_ref[...] + 1e-5) * gam_ref[...] + bet_ref[...]
    h = jnp.maximum(h, 0.0)
    if residual:
        h = h + x
    xo_ref[...] = h


def _pool_body(x_ref, b_ref, sum_ref, cnt_ref, max_ref):
    @pl.when(pl.program_id(0) == 0)
    def _():
        sum_ref[...] = jnp.zeros_like(sum_ref)
        cnt_ref[...] = jnp.zeros_like(cnt_ref)
        max_ref[...] = jnp.full_like(max_ref, -jnp.inf)

    x = x_ref[...]
    b = b_ref[...]
    onehot = (b == lax.broadcasted_iota(jnp.int32, (BN, NG), 1)).astype(jnp.float32)
    dn = (((0,), (0,)), ((), ()))
    sum_ref[...] += lax.dot_general(onehot, x, dn,
                                    preferred_element_type=jnp.float32)
    cnt_ref[...] += lax.dot_general(onehot, jnp.ones((BN, HID), jnp.float32), dn,
                                    preferred_element_type=jnp.float32)

    def gbody(g, c):
        masked = jnp.where(b == g, x, -jnp.inf)
        m = jnp.max(masked, axis=0, keepdims=True)
        max_ref[pl.ds(g, 1), :] = jnp.maximum(max_ref[pl.ds(g, 1), :], m)
        return c

    lax.fori_loop(0, NG, gbody, 0)


def _mlp_body(sum_ref, cnt_ref, max_ref, f1_ref, f1b_ref, g1_ref, be1_ref,
              m1_ref, v1_ref, f2_ref, f2b_ref, g2_ref, be2_ref, m2_ref, v2_ref,
              f3_ref, b3_ref, wh_ref, bh_ref, out_ref):
    s = sum_ref[...]
    c = jnp.maximum(cnt_ref[...], 1.0)
    mx = max_ref[...]
    h = jnp.concatenate([s / c, mx, s], axis=1)
    h = _dot(h, f1_ref[...]) + f1b_ref[...]
    h = (h - m1_ref[...]) / jnp.sqrt(v1_ref[...] + 1e-5) * g1_ref[...] + be1_ref[...]
    h = jnp.maximum(h, 0.0)
    h = _dot(h, f2_ref[...]) + f2b_ref[...]
    h = (h - m2_ref[...]) / jnp.sqrt(v2_ref[...] + 1e-5) * g2_ref[...] + be2_ref[...]
    h = jnp.maximum(h, 0.0)
    h = jnp.maximum(_dot(h, f3_ref[...]) + b3_ref[...], 0.0)
    out_ref[...] = _dot(h, wh_ref[...]) + bh_ref[...]


def _full(shape):
    return pl.BlockSpec(shape, lambda *_: tuple(0 for _ in shape))


def _rows(shape):
    n = len(shape)
    return pl.BlockSpec(shape, lambda i: (i,) + (0,) * (n - 1))


def _tc_init(x_pad, wemb, bemb):
    return pl.pallas_call(
        _init_body,
        grid=(NP_ // BN,),
        in_specs=[_rows((BN, 16)), _full((16, HID)), _full((1, HID))],
        out_specs=_rows((BN, HID)),
        out_shape=jax.ShapeDtypeStruct((NP_, HID), jnp.float32),
    )(x_pad, wemb, bemb)


def _tc_edge(ea_pad, xd, xs, wemb, bemb, w, b):
    return pl.pallas_call(
        _edge_body,
        grid=(EP_ // BE,),
        in_specs=[_rows((BE, 8)), _rows((BE, HID)), _rows((BE, HID)),
                  _full((8, HID)), _full((1, HID)),
                  _full((3 * HID, 256)), _full((1, 256))],
        out_specs=_rows((BE, HID)),
        out_shape=jax.ShapeDtypeStruct((EP_, HID), jnp.float32),
    )(ea_pad, xd, xs, wemb, bemb, w, b)


def _tc_node(x, a0, a1, gam, bet, mu, var, residual):
    return pl.pallas_call(
        functools.partial(_node_body, residual=residual),
        grid=(NP_ // BN,),
        in_specs=[_rows((BN, HID)), _rows((BN, HID)), _rows((BN, HID))] +
                 [_full((1, HID))] * 4,
        out_specs=_rows((BN, HID)),
        out_shape=jax.ShapeDtypeStruct((NP_, HID), jnp.float32),
    )(x, a0, a1, gam, bet, mu, var)


def _tc_pool(x, batch_pad):
    return pl.pallas_call(
        _pool_body,
        grid=(NP_ // BN,),
        in_specs=[_rows((BN, HID)), _rows((BN, 1))],
        out_specs=[_full((NG, HID)), _full((NG, HID)), _full((NG, HID))],
        out_shape=[jax.ShapeDtypeStruct((NG, HID), jnp.float32),
                   jax.ShapeDtypeStruct((NG, HID), jnp.float32),
                   jax.ShapeDtypeStruct((NG, HID), jnp.float32)],
    )(x, batch_pad)


def _tc_mlp(s, c, mx, *weights):
    return pl.pallas_call(
        _mlp_body,
        in_specs=[_full((NG, HID))] * 3 + [
            _full((3 * HID, 256)), _full((1, 256)),
            _full((1, 256)), _full((1, 256)), _full((1, 256)), _full((1, 256)),
            _full((256, HID)), _full((1, HID)),
            _full((1, HID)), _full((1, HID)), _full((1, HID)), _full((1, HID)),
            _full((HID, NG)), _full((1, NG)),
            _full((NG, 4)), _full((1, 4))],
        out_specs=_full((NG, 4)),
        out_shape=jax.ShapeDtypeStruct((NG, 4), jnp.float32),
    )(s, c, mx, *weights)


@jax.jit
def _run(x, edge_attr, edge_index, batch, params):
    f32 = jnp.float32
    w_conv = jnp.concatenate([params['conv_wf'], params['conv_ws']], axis=2)
    b_conv = jnp.concatenate([params['conv_bf'], params['conv_bs']], axis=1)

    node_wemb = jnp.zeros((16, HID), f32).at[:10].set(params['node_emb_w'])
    node_bemb = params['node_emb_b'][None, :]
    edge_wemb = jnp.zeros((8, HID), f32).at[:3].set(params['edge_emb_w'])
    edge_bemb = params['edge_emb_b'][None, :]

    def row(v):
        return v[None, :]

    wh = jnp.concatenate([params['head_voltage_w'], params['head_energy_w'],
                          params['head_density_w'], params['head_hull_w']], axis=1)
    bh = jnp.concatenate([params['head_voltage_b'], params['head_energy_b'],
                          params['head_density_b'], params['head_hull_b']])[None, :]

    x_pad = jnp.zeros((NP_, 16), f32).at[:NN, :10].set(x)
    ea_pad = jnp.zeros((EP_, 8), f32).at[:NE, :3].set(edge_attr)
    dst = jnp.full((EP_,), NN, jnp.int32).at[:NE].set(edge_index[1])
    src = jnp.zeros((EP_,), jnp.int32).at[:NE].set(edge_index[0])
    batch_pad = jnp.full((NP_, 1), NG, jnp.int32).at[:NN, 0].set(batch)
    zeros_nodes = jnp.zeros((NP_, HID), f32)

    xh = _tc_init(x_pad, node_wemb, node_bemb)
    for i in range(10):
        xd, xs = _sc_gather(xh, dst, src)
        msg = _tc_edge(ea_pad, xd, xs, edge_wemb, edge_bemb,
                       w_conv[i], b_conv[i][None, :])
        agg = _sc_scatter(msg, dst, zeros_nodes)
        xh = _tc_node(xh, agg[:NP_], agg[NP_:],
                      row(params['bn_gamma'][i]), row(params['bn_beta'][i]),
                      row(params['bn_mean'][i]), row(params['bn_var'][i]),
                      residual=(i % 2 == 1))

    s, c, mx = _tc_pool(xh, batch_pad)
    out = _tc_mlp(s, c, mx,
                  params['fc1_w'], row(params['fc1_b']),
                  row(params['bn1_gamma']), row(params['bn1_beta']),
                  row(params['bn1_mean']), row(params['bn1_var']),
                  params['fc2_w'], row(params['fc2_b']),
                  row(params['bn2_gamma']), row(params['bn2_beta']),
                  row(params['bn2_mean']), row(params['bn2_var']),
                  params['fc3_w'], row(params['fc3_b']), wh, bh)
    return out[:, 0:1], out[:, 1:2], out[:, 2:3], out[:, 3:4]


def kernel(x, edge_attr, edge_index, batch, params):
    return _run(x, edge_attr, edge_index, batch, params)
